# Initial kernel scaffold; baseline (speedup 1.0000x reference)
import jax
import jax.numpy as jnp
from jax.experimental import pallas as pl


def kernel(unconditioned, codes, codebook, W_proj, b_proj, gate):
    raise NotImplementedError("write your pallas kernel here")


# speedup vs baseline: 1.4597x; 1.4597x over previous
import functools

import jax
import jax.numpy as jnp
from jax import lax
from jax.experimental import pallas as pl
from jax.experimental.pallas import tpu as pltpu
from jax.experimental.pallas import tpu_sc as plsc

_B, _S, _H = 4, 8192, 1024
_K, _D = 8192, 64
_N = _B * _S

_NC, _NS = 2, 16
_NW = _NC * _NS
_ROWS_PER_W = _N // _NW
_DP = 128
_CHUNK = 512

_TOK_BLOCK = 512


def _sc_gather(table_padded, codes_flat):
    mesh = plsc.VectorSubcoreMesh(core_axis_name="c", subcore_axis_name="s")

    @functools.partial(
        pl.kernel,
        mesh=mesh,
        out_type=jax.ShapeDtypeStruct((_N, _DP), jnp.float32),
        scratch_types=[
            pltpu.VMEM((_ROWS_PER_W,), jnp.int32),
            pltpu.VMEM((_CHUNK, _DP), jnp.float32),
            pltpu.SemaphoreType.DMA,
        ],
    )
    def gather_kernel(table_hbm, idx_hbm, out_hbm, idx_v, rows_v, sem):
        wid = lax.axis_index("s") * _NC + lax.axis_index("c")
        base = wid * _ROWS_PER_W
        pltpu.sync_copy(idx_hbm.at[pl.ds(base, _ROWS_PER_W)], idx_v)

        @pl.loop(0, _ROWS_PER_W, step=_CHUNK)
        def _(r):
            pltpu.async_copy(
                table_hbm.at[idx_v.at[pl.ds(r, _CHUNK)]], rows_v, sem
            ).wait()
            pltpu.sync_copy(rows_v, out_hbm.at[pl.ds(base + r, _CHUNK)])

    return gather_kernel(table_padded, codes_flat)


def _tc_body(uncond_ref, embs_ref, w_ref, b_ref, g_ref, out_ref):
    g = g_ref[...]
    proj = jnp.dot(embs_ref[...], w_ref[...],
                   preferred_element_type=jnp.float32)
    out_ref[...] = uncond_ref[...] + (proj + b_ref[...]) * g


def _tc_fused(uncond2d, embs, W_proj, b_proj2d, gate):
    grid = (_N // _TOK_BLOCK,)
    return pl.pallas_call(
        _tc_body,
        grid=grid,
        in_specs=[
            pl.BlockSpec((_TOK_BLOCK, _H), lambda i: (i, 0)),
            pl.BlockSpec((_TOK_BLOCK, _DP), lambda i: (i, 0)),
            pl.BlockSpec((_DP, _H), lambda i: (0, 0)),
            pl.BlockSpec((1, _H), lambda i: (0, 0)),
            pl.BlockSpec((1, _H), lambda i: (0, 0)),
        ],
        out_specs=pl.BlockSpec((_TOK_BLOCK, _H), lambda i: (i, 0)),
        out_shape=jax.ShapeDtypeStruct((_N, _H), jnp.float32),
        compiler_params=pltpu.CompilerParams(
            dimension_semantics=("parallel",),
        ),
    )(uncond2d, embs, W_proj, b_proj2d, gate)


def kernel(unconditioned, codes, codebook, W_proj, b_proj, gate):
    codes_flat = codes.reshape(_N)
    table_padded = jnp.pad(codebook, ((0, 0), (0, _DP - _D)))
    w_padded = jnp.pad(W_proj, ((0, _DP - _D), (0, 0)))
    embs = _sc_gather(table_padded, codes_flat)
    uncond2d = unconditioned.reshape(_N, _H)
    out = _tc_fused(uncond2d, embs, w_padded, b_proj.reshape(1, _H), gate)
    return out.reshape(_B, _S, _H)


# speedup vs baseline: 1.4629x; 1.0022x over previous
import functools

import jax
import jax.numpy as jnp
from jax import lax
from jax.experimental import pallas as pl
from jax.experimental.pallas import tpu as pltpu
from jax.experimental.pallas import tpu_sc as plsc

_B, _S, _H = 4, 8192, 1024
_K, _D = 8192, 64
_N = _B * _S

_NC, _NS = 2, 16
_NW = _NC * _NS
_ROWS_PER_W = _N // _NW
_DP = 128
_CHUNK = 512

_TOK_BLOCK = 512


def _sc_gather(table_padded, codes_flat):
    mesh = plsc.VectorSubcoreMesh(core_axis_name="c", subcore_axis_name="s")

    @functools.partial(
        pl.kernel,
        mesh=mesh,
        out_type=jax.ShapeDtypeStruct((_N, _DP), jnp.float32),
        scratch_types=[
            pltpu.VMEM((_ROWS_PER_W,), jnp.int32),
            pltpu.VMEM((_CHUNK, _DP), jnp.float32),
            pltpu.SemaphoreType.DMA,
        ],
    )
    def gather_kernel(table_hbm, idx_hbm, out_hbm, idx_v, rows_v, sem):
        wid = lax.axis_index("s") * _NC + lax.axis_index("c")
        base = wid * _ROWS_PER_W
        pltpu.sync_copy(idx_hbm.at[pl.ds(base, _ROWS_PER_W)], idx_v)

        @pl.loop(0, _ROWS_PER_W, step=_CHUNK)
        def _(r):
            pltpu.async_copy(
                table_hbm.at[idx_v.at[pl.ds(r, _CHUNK)]], rows_v, sem
            ).wait()
            pltpu.sync_copy(rows_v, out_hbm.at[pl.ds(base + r, _CHUNK)])

    return gather_kernel(table_padded, codes_flat)


def _tc_body(uncond_ref, embs_ref, w_ref, b_ref, g_ref, out_ref):
    g = g_ref[...]
    proj = jnp.dot(embs_ref[...].astype(jnp.bfloat16),
                   w_ref[...].astype(jnp.bfloat16),
                   preferred_element_type=jnp.float32)
    out_ref[...] = uncond_ref[...] + (proj + b_ref[...]) * g


def _tc_fused(uncond2d, embs, W_proj, b_proj2d, gate):
    grid = (_N // _TOK_BLOCK,)
    return pl.pallas_call(
        _tc_body,
        grid=grid,
        in_specs=[
            pl.BlockSpec((_TOK_BLOCK, _H), lambda i: (i, 0)),
            pl.BlockSpec((_TOK_BLOCK, _DP), lambda i: (i, 0)),
            pl.BlockSpec((_DP, _H), lambda i: (0, 0)),
            pl.BlockSpec((1, _H), lambda i: (0, 0)),
            pl.BlockSpec((1, _H), lambda i: (0, 0)),
        ],
        out_specs=pl.BlockSpec((_TOK_BLOCK, _H), lambda i: (i, 0)),
        out_shape=jax.ShapeDtypeStruct((_N, _H), jnp.float32),
        compiler_params=pltpu.CompilerParams(
            dimension_semantics=("parallel",),
        ),
    )(uncond2d, embs, W_proj, b_proj2d, gate)


def kernel(unconditioned, codes, codebook, W_proj, b_proj, gate):
    codes_flat = codes.reshape(_N)
    table_padded = jnp.pad(codebook, ((0, 0), (0, _DP - _D)))
    w_padded = jnp.pad(W_proj, ((0, _DP - _D), (0, 0)))
    embs = _sc_gather(table_padded, codes_flat)
    uncond2d = unconditioned.reshape(_N, _H)
    out = _tc_fused(uncond2d, embs, w_padded, b_proj.reshape(1, _H), gate)
    return out.reshape(_B, _S, _H)


# speedup vs baseline: 1.5970x; 1.0917x over previous
import functools

import jax
import jax.numpy as jnp
from jax import lax
from jax.experimental import pallas as pl
from jax.experimental.pallas import tpu as pltpu
from jax.experimental.pallas import tpu_sc as plsc

_B, _S, _H = 4, 8192, 1024
_K, _D = 8192, 64
_N = _B * _S

_NC, _NS = 2, 16
_NW = _NC * _NS
_ROWS_PER_W = _N // _NW
_DP = 128
_CHUNK = 512

_TOK_BLOCK = 1024


def _sc_gather(table_padded, codes_flat):
    mesh = plsc.VectorSubcoreMesh(core_axis_name="c", subcore_axis_name="s")

    @functools.partial(
        pl.kernel,
        mesh=mesh,
        out_type=jax.ShapeDtypeStruct((_N, _DP), jnp.float32),
        scratch_types=[
            pltpu.VMEM((_ROWS_PER_W,), jnp.int32),
            pltpu.VMEM((_CHUNK, _DP), jnp.float32),
            pltpu.SemaphoreType.DMA,
        ],
    )
    def gather_kernel(table_hbm, idx_hbm, out_hbm, idx_v, rows_v, sem):
        wid = lax.axis_index("s") * _NC + lax.axis_index("c")
        base = wid * _ROWS_PER_W
        pltpu.sync_copy(idx_hbm.at[pl.ds(base, _ROWS_PER_W)], idx_v)

        @pl.loop(0, _ROWS_PER_W, step=_CHUNK)
        def _(r):
            pltpu.async_copy(
                table_hbm.at[idx_v.at[pl.ds(r, _CHUNK)]], rows_v, sem
            ).wait()
            pltpu.sync_copy(rows_v, out_hbm.at[pl.ds(base + r, _CHUNK)])

    return gather_kernel(table_padded, codes_flat)


def _tc_body(uncond_ref, embs_ref, w_ref, b_ref, g_ref, out_ref):
    g = g_ref[...]
    proj = jnp.dot(embs_ref[...].astype(jnp.bfloat16),
                   w_ref[...].astype(jnp.bfloat16),
                   preferred_element_type=jnp.float32)
    out_ref[...] = uncond_ref[...] + (proj + b_ref[...]) * g


def _tc_fused(uncond2d, embs, W_proj, b_proj2d, gate):
    grid = (_N // _TOK_BLOCK,)
    return pl.pallas_call(
        _tc_body,
        grid=grid,
        in_specs=[
            pl.BlockSpec((_TOK_BLOCK, _H), lambda i: (i, 0)),
            pl.BlockSpec((_TOK_BLOCK, _DP), lambda i: (i, 0)),
            pl.BlockSpec((_DP, _H), lambda i: (0, 0)),
            pl.BlockSpec((1, _H), lambda i: (0, 0)),
            pl.BlockSpec((1, _H), lambda i: (0, 0)),
        ],
        out_specs=pl.BlockSpec((_TOK_BLOCK, _H), lambda i: (i, 0)),
        out_shape=jax.ShapeDtypeStruct((_N, _H), jnp.float32),
        compiler_params=pltpu.CompilerParams(
            dimension_semantics=("parallel",),
        ),
    )(uncond2d, embs, W_proj, b_proj2d, gate)


def kernel(unconditioned, codes, codebook, W_proj, b_proj, gate):
    codes_flat = codes.reshape(_N)
    table_padded = jnp.pad(codebook, ((0, 0), (0, _DP - _D)))
    w_padded = jnp.pad(W_proj, ((0, _DP - _D), (0, 0)))
    embs = _sc_gather(table_padded, codes_flat)
    uncond2d = unconditioned.reshape(_N, _H)
    out = _tc_fused(uncond2d, embs, w_padded, b_proj.reshape(1, _H), gate)
    return out.reshape(_B, _S, _H)


# speedup vs baseline: 1.6213x; 1.0152x over previous
import functools

import jax
import jax.numpy as jnp
from jax import lax
from jax.experimental import pallas as pl
from jax.experimental.pallas import tpu as pltpu
from jax.experimental.pallas import tpu_sc as plsc

_B, _S, _H = 4, 8192, 1024
_K, _D = 8192, 64
_N = _B * _S

_NC, _NS = 2, 16
_NW = _NC * _NS
_ROWS_PER_W = _N // _NW
_DP = 128
_CHUNK = 512

_TOK_BLOCK = 2048


def _sc_gather(table_padded, codes_flat):
    mesh = plsc.VectorSubcoreMesh(core_axis_name="c", subcore_axis_name="s")

    @functools.partial(
        pl.kernel,
        mesh=mesh,
        out_type=jax.ShapeDtypeStruct((_N, _DP), jnp.float32),
        scratch_types=[
            pltpu.VMEM((_ROWS_PER_W,), jnp.int32),
            pltpu.VMEM((_CHUNK, _DP), jnp.float32),
            pltpu.SemaphoreType.DMA,
        ],
    )
    def gather_kernel(table_hbm, idx_hbm, out_hbm, idx_v, rows_v, sem):
        wid = lax.axis_index("s") * _NC + lax.axis_index("c")
        base = wid * _ROWS_PER_W
        pltpu.sync_copy(idx_hbm.at[pl.ds(base, _ROWS_PER_W)], idx_v)

        @pl.loop(0, _ROWS_PER_W, step=_CHUNK)
        def _(r):
            pltpu.async_copy(
                table_hbm.at[idx_v.at[pl.ds(r, _CHUNK)]], rows_v, sem
            ).wait()
            pltpu.sync_copy(rows_v, out_hbm.at[pl.ds(base + r, _CHUNK)])

    return gather_kernel(table_padded, codes_flat)


def _tc_body(uncond_ref, embs_ref, w_ref, b_ref, g_ref, out_ref):
    g = g_ref[...]
    proj = jnp.dot(embs_ref[...].astype(jnp.bfloat16),
                   w_ref[...].astype(jnp.bfloat16),
                   preferred_element_type=jnp.float32)
    out_ref[...] = uncond_ref[...] + (proj + b_ref[...]) * g


def _tc_fused(uncond2d, embs, W_proj, b_proj2d, gate):
    grid = (_N // _TOK_BLOCK,)
    return pl.pallas_call(
        _tc_body,
        grid=grid,
        in_specs=[
            pl.BlockSpec((_TOK_BLOCK, _H), lambda i: (i, 0)),
            pl.BlockSpec((_TOK_BLOCK, _DP), lambda i: (i, 0)),
            pl.BlockSpec((_DP, _H), lambda i: (0, 0)),
            pl.BlockSpec((1, _H), lambda i: (0, 0)),
            pl.BlockSpec((1, _H), lambda i: (0, 0)),
        ],
        out_specs=pl.BlockSpec((_TOK_BLOCK, _H), lambda i: (i, 0)),
        out_shape=jax.ShapeDtypeStruct((_N, _H), jnp.float32),
        compiler_params=pltpu.CompilerParams(
            dimension_semantics=("parallel",),
        ),
    )(uncond2d, embs, W_proj, b_proj2d, gate)


def kernel(unconditioned, codes, codebook, W_proj, b_proj, gate):
    codes_flat = codes.reshape(_N)
    table_padded = jnp.pad(codebook, ((0, 0), (0, _DP - _D)))
    w_padded = jnp.pad(W_proj, ((0, _DP - _D), (0, 0)))
    embs = _sc_gather(table_padded, codes_flat)
    uncond2d = unconditioned.reshape(_N, _H)
    out = _tc_fused(uncond2d, embs, w_padded, b_proj.reshape(1, _H), gate)
    return out.reshape(_B, _S, _H)
